# Initial kernel scaffold; baseline (speedup 1.0000x reference)
import jax
import jax.numpy as jnp
from jax.experimental import pallas as pl


def kernel(positions, pe):
    raise NotImplementedError("write your pallas kernel here")


# speedup vs baseline: 2.3137x; 2.3137x over previous
import functools

import jax
import jax.numpy as jnp
from jax import lax
from jax.experimental import pallas as pl
from jax.experimental.pallas import tpu as pltpu
from jax.experimental.pallas import tpu_sc as plsc


@functools.lru_cache(maxsize=None)
def _build_gather(B, D, chunk):
    info = plsc.get_sparse_core_info()
    NC, NS = info.num_cores, info.num_subcores
    NW = NC * NS
    assert B % (NW * chunk) == 0
    b_per_w = B // NW
    n_chunks = b_per_w // chunk
    mesh = plsc.VectorSubcoreMesh(core_axis_name="c", subcore_axis_name="s")

    @functools.partial(
        pl.kernel,
        mesh=mesh,
        out_type=jax.ShapeDtypeStruct((B, D), jnp.float32),
        scratch_types=[
            pltpu.VMEM((n_chunks, chunk), jnp.int32),
            pltpu.VMEM((chunk, D), jnp.float32),
            pltpu.VMEM((chunk, D), jnp.float32),
            pltpu.SemaphoreType.DMA,
            pltpu.SemaphoreType.DMA,
        ],
    )
    def k(idx_hbm, table_hbm, out_hbm, idx_v, buf0, buf1, sem0, sem1):
        wid = lax.axis_index("s") * NC + lax.axis_index("c")
        base = wid * b_per_w
        pltpu.sync_copy(idx_hbm.at[wid], idx_v)
        bufs = (buf0, buf1)
        sems = (sem0, sem1)
        copies = [None, None]
        copies[0] = pltpu.async_copy(table_hbm.at[idx_v.at[0]], bufs[0], sems[0])
        for g in range(n_chunks):
            cur = g % 2
            nxt = (g + 1) % 2
            if g + 1 < n_chunks:
                copies[nxt] = pltpu.async_copy(
                    table_hbm.at[idx_v.at[g + 1]], bufs[nxt], sems[nxt]
                )
            copies[cur].wait()
            pltpu.sync_copy(bufs[cur], out_hbm.at[pl.ds(base + g * chunk, chunk)])

    return k, NW, n_chunks, chunk


def kernel(positions, pe):
    Bb, S = positions.shape
    V, D = pe.shape
    B = Bb * S
    chunk = 32
    k, NW, n_chunks, chunk = _build_gather(B, D, chunk)
    idx = positions.reshape(NW, n_chunks, chunk).astype(jnp.int32)
    out = k(idx, pe)
    return out.reshape(Bb, S, D)


# speedup vs baseline: 2.3205x; 1.0029x over previous
import functools

import jax
import jax.numpy as jnp
from jax import lax
from jax.experimental import pallas as pl
from jax.experimental.pallas import tpu as pltpu
from jax.experimental.pallas import tpu_sc as plsc


@functools.lru_cache(maxsize=None)
def _build_gather(B, D, chunk):
    info = plsc.get_sparse_core_info()
    NC, NS = info.num_cores, info.num_subcores
    NW = NC * NS
    assert B % (NW * chunk) == 0
    b_per_w = B // NW
    n_chunks = b_per_w // chunk
    mesh = plsc.VectorSubcoreMesh(core_axis_name="c", subcore_axis_name="s")

    nbuf = 3

    @functools.partial(
        pl.kernel,
        mesh=mesh,
        out_type=jax.ShapeDtypeStruct((B, D), jnp.float32),
        scratch_types=[
            pltpu.VMEM((n_chunks, chunk), jnp.int32),
        ]
        + [pltpu.VMEM((chunk, D), jnp.float32) for _ in range(nbuf)]
        + [pltpu.SemaphoreType.DMA for _ in range(2 * nbuf)],
    )
    def k(idx_hbm, table_hbm, out_hbm, idx_v, *rest):
        bufs = rest[:nbuf]
        gsems = rest[nbuf : 2 * nbuf]
        ssems = rest[2 * nbuf :]
        wid = lax.axis_index("s") * NC + lax.axis_index("c")
        base = wid * b_per_w
        pltpu.sync_copy(idx_hbm.at[wid], idx_v)
        gcp = [None] * nbuf
        scp = [None] * nbuf
        for step in range(n_chunks + nbuf - 1):
            slot = step % nbuf
            if step < n_chunks:
                if step >= nbuf:
                    scp[slot].wait()
                gcp[slot] = pltpu.async_copy(
                    table_hbm.at[idx_v.at[step]], bufs[slot], gsems[slot]
                )
            g = step - (nbuf - 1)
            if g >= 0:
                gslot = g % nbuf
                gcp[gslot].wait()
                scp[gslot] = pltpu.async_copy(
                    bufs[gslot],
                    out_hbm.at[pl.ds(base + g * chunk, chunk)],
                    ssems[gslot],
                )
        for g in range(max(0, n_chunks - nbuf), n_chunks):
            scp[g % nbuf].wait()

    return k, NW, n_chunks, chunk


def kernel(positions, pe):
    Bb, S = positions.shape
    V, D = pe.shape
    B = Bb * S
    chunk = 32
    k, NW, n_chunks, chunk = _build_gather(B, D, chunk)
    idx = positions.reshape(NW, n_chunks, chunk).astype(jnp.int32)
    out = k(idx, pe)
    return out.reshape(Bb, S, D)


# speedup vs baseline: 2.3221x; 1.0007x over previous
import functools

import jax
import jax.numpy as jnp
from jax import lax
from jax.experimental import pallas as pl
from jax.experimental.pallas import tpu as pltpu
from jax.experimental.pallas import tpu_sc as plsc


@functools.lru_cache(maxsize=None)
def _build_gather(B, D, chunk):
    info = plsc.get_sparse_core_info()
    NC, NS = info.num_cores, info.num_subcores
    NW = NC * NS
    assert B % (NW * chunk) == 0
    b_per_w = B // NW
    n_chunks = b_per_w // chunk
    mesh = plsc.VectorSubcoreMesh(core_axis_name="c", subcore_axis_name="s")

    nbuf = 6

    @functools.partial(
        pl.kernel,
        mesh=mesh,
        out_type=jax.ShapeDtypeStruct((B, D), jnp.float32),
        scratch_types=[
            pltpu.VMEM((n_chunks, chunk), jnp.int32),
        ]
        + [pltpu.VMEM((chunk, D), jnp.float32) for _ in range(nbuf)]
        + [pltpu.SemaphoreType.DMA for _ in range(2 * nbuf)],
    )
    def k(idx_hbm, table_hbm, out_hbm, idx_v, *rest):
        bufs = rest[:nbuf]
        gsems = rest[nbuf : 2 * nbuf]
        ssems = rest[2 * nbuf :]
        wid = lax.axis_index("s") * NC + lax.axis_index("c")
        base = wid * b_per_w
        pltpu.sync_copy(idx_hbm.at[wid], idx_v)
        gcp = [None] * nbuf
        scp = [None] * nbuf
        for step in range(n_chunks + nbuf - 1):
            slot = step % nbuf
            if step < n_chunks:
                if step >= nbuf:
                    scp[slot].wait()
                gcp[slot] = pltpu.async_copy(
                    table_hbm.at[idx_v.at[step]], bufs[slot], gsems[slot]
                )
            g = step - (nbuf - 1)
            if g >= 0:
                gslot = g % nbuf
                gcp[gslot].wait()
                scp[gslot] = pltpu.async_copy(
                    bufs[gslot],
                    out_hbm.at[pl.ds(base + g * chunk, chunk)],
                    ssems[gslot],
                )
        for g in range(max(0, n_chunks - nbuf), n_chunks):
            scp[g % nbuf].wait()

    return k, NW, n_chunks, chunk


def kernel(positions, pe):
    Bb, S = positions.shape
    V, D = pe.shape
    B = Bb * S
    chunk = 16
    k, NW, n_chunks, chunk = _build_gather(B, D, chunk)
    idx = positions.reshape(NW, n_chunks, chunk).astype(jnp.int32)
    out = k(idx, pe)
    return out.reshape(Bb, S, D)
